# baseline (speedup 1.0000x reference)
import functools

import jax
import jax.numpy as jnp
from jax import lax
from jax.experimental import pallas as pl
from jax.experimental.pallas import tpu as pltpu
from jax.experimental.pallas import tpu_sc as plsc

B, L, D = 4096, 200, 64
N = B * L
NC, NS = 2, 16
NW = NC * NS
ROWS_PER_W = N // NW
GW = 128
SUB = 4
KW = GW * SUB
LANES = 16


def _sc_encode(xf, im, isec, ih, idow, t1):
    mesh = plsc.VectorSubcoreMesh(core_axis_name="c", subcore_axis_name="s")

    @functools.partial(
        pl.kernel,
        out_type=jax.ShapeDtypeStruct((N, D), jnp.float32),
        mesh=mesh,
        scratch_types=[
            pltpu.VMEM((KW, D), jnp.float32),
            pltpu.VMEM((KW, D), jnp.float32),
            pltpu.VMEM((KW,), jnp.int32),
            pltpu.VMEM((KW,), jnp.int32),
            pltpu.VMEM((KW,), jnp.int32),
            pltpu.VMEM((KW,), jnp.int32),
            pltpu.VMEM((KW,), jnp.int32),
            pltpu.VMEM((KW,), jnp.int32),
            pltpu.VMEM((KW,), jnp.int32),
            pltpu.VMEM((KW,), jnp.int32),
            pltpu.VMEM((KW,), jnp.int32),
            pltpu.VMEM((KW,), jnp.int32),
            pltpu.SemaphoreType.DMA,
            pltpu.SemaphoreType.DMA,
            pltpu.SemaphoreType.DMA,
            pltpu.SemaphoreType.DMA,
            pltpu.SemaphoreType.DMA,
            pltpu.SemaphoreType.DMA,
        ],
        compiler_params=pltpu.CompilerParams(use_tc_tiling_on_sc=False),
    )
    def k(x_hbm, im_hbm, is_hbm, ih_hbm, id_hbm, t1_hbm, o_hbm,
          acc0, acc1, im0, is0, ih0, id0, im1, is1, ih1, id1,
          c1v0, c1v1, sl0, sl1, sa0, sa1, so0, so1):
        wid = lax.axis_index("s") * NC + lax.axis_index("c")
        base = wid * ROWS_PER_W
        accs = (acc0, acc1)
        imvs, isvs = (im0, im1), (is0, is1)
        ihvs, idvs = (ih0, ih1), (id0, id1)
        c1vs = (c1v0, c1v1)
        sls, sas, sos = (sl0, sl1), (sa0, sa1), (so0, so1)

        def fire_loads(b, off):
            pltpu.async_copy(x_hbm.at[pl.ds(off, KW)], accs[b], sls[b])
            pltpu.async_copy(im_hbm.at[pl.ds(off, KW)], imvs[b], sls[b])
            pltpu.async_copy(is_hbm.at[pl.ds(off, KW)], isvs[b], sls[b])
            pltpu.async_copy(ih_hbm.at[pl.ds(off, KW)], ihvs[b], sls[b])
            pltpu.async_copy(id_hbm.at[pl.ds(off, KW)], idvs[b], sls[b])

        def wait_loads(b):
            pltpu.make_async_copy(x_hbm.at[pl.ds(0, KW)], accs[b], sls[b]).wait()
            pltpu.make_async_copy(im_hbm.at[pl.ds(0, KW)], imvs[b], sls[b]).wait()
            pltpu.make_async_copy(is_hbm.at[pl.ds(0, KW)], isvs[b], sls[b]).wait()
            pltpu.make_async_copy(ih_hbm.at[pl.ds(0, KW)], ihvs[b], sls[b]).wait()
            pltpu.make_async_copy(id_hbm.at[pl.ds(0, KW)], idvs[b], sls[b]).wait()

        def combine_indices(b):
            for j in range(0, KW, LANES):
                s = pl.ds(j, LANES)
                c1vs[b].at[s][...] = (
                    (ihvs[b].at[s][...] * 8 + idvs[b].at[s][...]) * 3721
                    + imvs[b].at[s][...] * 61 + isvs[b].at[s][...])

        def fire_adds(b):
            for j in range(SUB):
                w = pl.ds(j * GW, GW)
                pltpu.async_copy(t1_hbm.at[c1vs[b].at[w]], accs[b].at[w],
                                 sas[b], add=True)

        def wait_adds(b):
            for j in range(SUB):
                w = pl.ds(j * GW, GW)
                pltpu.make_async_copy(t1_hbm.at[c1vs[b].at[w]], accs[b].at[w],
                                      sas[b]).wait()

        def fire_store(b, off):
            pltpu.async_copy(accs[b], o_hbm.at[pl.ds(off, KW)], sos[b])

        def wait_store(b):
            pltpu.make_async_copy(accs[b], o_hbm.at[pl.ds(0, KW)], sos[b]).wait()

        fire_loads(0, base)
        fire_loads(1, base + KW)

        @pl.loop(0, ROWS_PER_W, step=2 * KW)
        def _(r):
            off_a = base + r
            off_b = off_a + KW
            off_c = off_b + KW
            off_d = off_c + KW

            wait_loads(0)
            combine_indices(0)
            fire_adds(0)

            wait_loads(1)
            combine_indices(1)
            fire_adds(1)

            wait_adds(0)
            fire_store(0, off_a)

            wait_store(0)

            @pl.when(r + 2 * KW < ROWS_PER_W)
            def _():
                fire_loads(0, off_c)

            wait_adds(1)
            fire_store(1, off_b)
            wait_store(1)

            @pl.when(r + 3 * KW < ROWS_PER_W)
            def _():
                fire_loads(1, off_d)

    return k(xf, im, isec, ih, idow, t1)


@jax.jit
def kernel(x, ts_hour, ts_minute, ts_second, ts_dayofweek,
           W_hour, W_minute, W_second, W_dayofweek):
    thd = (W_hour[:, None, :] + W_dayofweek[None, :, :]).reshape(200, 1, D)
    tms = (W_minute[:, None, :] + W_second[None, :, :]).reshape(1, 3721, D)
    t1 = (thd + tms).reshape(200 * 3721, D)
    xf = x.reshape(N, D)
    im = ts_minute.reshape(N)
    isec = ts_second.reshape(N)
    ih = ts_hour.reshape(N)
    idow = ts_dayofweek.reshape(N)
    out = _sc_encode(xf, im, isec, ih, idow, t1)
    return out.reshape(B, L, D)


# speedup vs baseline: 1.0614x; 1.0614x over previous
import functools

import jax
import jax.numpy as jnp
from jax import lax
from jax.experimental import pallas as pl
from jax.experimental.pallas import tpu as pltpu
from jax.experimental.pallas import tpu_sc as plsc

B, L, D = 4096, 200, 64
NC, NS = 2, 16
NW = NC * NS
BPW = B // NW
BB = 4
WINDOWS = ((0, 128), (128, 72))
LANES = 16


def _sc_encode(x, im, isec, ih, idow, t1, t2):
    mesh = plsc.VectorSubcoreMesh(core_axis_name="c", subcore_axis_name="s")

    idx_t = pltpu.VMEM((BB, L), jnp.int32)

    @functools.partial(
        pl.kernel,
        out_type=jax.ShapeDtypeStruct((B, L, D), jnp.float32),
        mesh=mesh,
        scratch_types=[
            pltpu.VMEM((BB, L, D), jnp.float32),
            pltpu.VMEM((BB, L, D), jnp.float32),
            idx_t, idx_t, idx_t, idx_t,
            idx_t, idx_t, idx_t, idx_t,
            idx_t, idx_t, idx_t, idx_t,
            pltpu.SemaphoreType.DMA,
            pltpu.SemaphoreType.DMA,
            pltpu.SemaphoreType.DMA,
            pltpu.SemaphoreType.DMA,
            pltpu.SemaphoreType.DMA,
            pltpu.SemaphoreType.DMA,
        ],
        compiler_params=pltpu.CompilerParams(use_tc_tiling_on_sc=False),
    )
    def k(x_hbm, im_hbm, is_hbm, ih_hbm, id_hbm, t1_hbm, t2_hbm, o_hbm,
          acc0, acc1, im0, is0, ih0, id0, im1, is1, ih1, id1,
          c1v0, c2v0, c1v1, c2v1, sl0, sl1, sa0, sa1, so0, so1):
        wid = lax.axis_index("s") * NC + lax.axis_index("c")
        base = wid * BPW
        accs = (acc0, acc1)
        imvs, isvs = (im0, im1), (is0, is1)
        ihvs, idvs = (ih0, ih1), (id0, id1)
        c1vs, c2vs = (c1v0, c1v1), (c2v0, c2v1)
        sls, sas, sos = (sl0, sl1), (sa0, sa1), (so0, so1)

        def fire_loads(b, off):
            pltpu.async_copy(x_hbm.at[pl.ds(off, BB)], accs[b], sls[b])
            pltpu.async_copy(im_hbm.at[pl.ds(off, BB)], imvs[b], sls[b])
            pltpu.async_copy(is_hbm.at[pl.ds(off, BB)], isvs[b], sls[b])
            pltpu.async_copy(ih_hbm.at[pl.ds(off, BB)], ihvs[b], sls[b])
            pltpu.async_copy(id_hbm.at[pl.ds(off, BB)], idvs[b], sls[b])

        def wait_loads(b):
            pltpu.make_async_copy(x_hbm.at[pl.ds(0, BB)], accs[b], sls[b]).wait()
            pltpu.make_async_copy(im_hbm.at[pl.ds(0, BB)], imvs[b], sls[b]).wait()
            pltpu.make_async_copy(is_hbm.at[pl.ds(0, BB)], isvs[b], sls[b]).wait()
            pltpu.make_async_copy(ih_hbm.at[pl.ds(0, BB)], ihvs[b], sls[b]).wait()
            pltpu.make_async_copy(id_hbm.at[pl.ds(0, BB)], idvs[b], sls[b]).wait()

        def combine_indices(b):
            for i in range(BB):
                for j in list(range(0, L - LANES, LANES)) + [L - LANES]:
                    s = (pl.ds(i, 1), pl.ds(j, LANES))
                    c1vs[b].at[*s][...] = (imvs[b].at[*s][...] * 61
                                           + isvs[b].at[*s][...])
                    c2vs[b].at[*s][...] = (ihvs[b].at[*s][...] * 8
                                           + idvs[b].at[*s][...])

        def fire_adds(b):
            for i in range(BB):
                for w0, wl in WINDOWS:
                    w = pl.ds(w0, wl)
                    pltpu.async_copy(t1_hbm.at[c1vs[b].at[i, w]],
                                     accs[b].at[i, w], sas[b], add=True)
                    pltpu.async_copy(t2_hbm.at[c2vs[b].at[i, w]],
                                     accs[b].at[i, w], sas[b], add=True)

        def wait_adds(b):
            for i in range(BB):
                for w0, wl in WINDOWS:
                    w = pl.ds(w0, wl)
                    pltpu.make_async_copy(t1_hbm.at[c1vs[b].at[i, w]],
                                          accs[b].at[i, w], sas[b]).wait()
                    pltpu.make_async_copy(t2_hbm.at[c2vs[b].at[i, w]],
                                          accs[b].at[i, w], sas[b]).wait()

        def fire_store(b, off):
            pltpu.async_copy(accs[b], o_hbm.at[pl.ds(off, BB)], sos[b])

        def wait_store(b):
            pltpu.make_async_copy(accs[b], o_hbm.at[pl.ds(0, BB)], sos[b]).wait()

        fire_loads(0, base)
        fire_loads(1, base + BB)

        @pl.loop(0, BPW, step=2 * BB)
        def _(r):
            off_a = base + r
            off_b = off_a + BB
            off_c = off_b + BB
            off_d = off_c + BB

            wait_loads(0)
            combine_indices(0)
            fire_adds(0)

            wait_loads(1)
            combine_indices(1)
            fire_adds(1)

            wait_adds(0)
            fire_store(0, off_a)

            wait_store(0)

            @pl.when(r + 2 * BB < BPW)
            def _():
                fire_loads(0, off_c)

            wait_adds(1)
            fire_store(1, off_b)
            wait_store(1)

            @pl.when(r + 3 * BB < BPW)
            def _():
                fire_loads(1, off_d)

    return k(x, im, isec, ih, idow, t1, t2)


@jax.jit
def kernel(x, ts_hour, ts_minute, ts_second, ts_dayofweek,
           W_hour, W_minute, W_second, W_dayofweek):
    t1 = (W_minute[:, None, :] + W_second[None, :, :]).reshape(61 * 61, D)
    t2 = (W_hour[:, None, :] + W_dayofweek[None, :, :]).reshape(25 * 8, D)
    return _sc_encode(x, ts_minute, ts_second, ts_hour, ts_dayofweek, t1, t2)
